# Initial kernel scaffold; baseline (speedup 1.0000x reference)
import jax
import jax.numpy as jnp
from jax.experimental import pallas as pl


def kernel(x, edge_index, W1, b1, W2, b2, W3, b3):
    raise NotImplementedError("write your pallas kernel here")


# speedup vs baseline: 17.3386x; 17.3386x over previous
import functools

import jax
import jax.numpy as jnp
from jax import lax
from jax.experimental import pallas as pl
from jax.experimental.pallas import tpu as pltpu
from jax.experimental.pallas import tpu_sc as plsc

N = 10000
E = 160000
D_IN = 256
HID = 128
D_OUT = 12

NC, NS = 2, 16
NW = NC * NS
N_PAD = 10240
E_PAD = 163840
C = 128
PER_W = E_PAD // NW
N_CHUNK = PER_W // C
ROWS_PER_TILE = N_PAD // NS
PAD_SPREAD = 240

_mesh = plsc.VectorSubcoreMesh(core_axis_name="c", subcore_axis_name="s",
                               num_cores=NC, num_subcores=NS)


def _zero_fill(zero_v, F):
  @pl.loop(0, 16)
  def _(r):
    @pl.loop(0, F, step=16)
    def _(j):
      zero_v.at[pl.ds(r, 1), pl.ds(j, 16)][...] = jnp.zeros((1, 16), jnp.float32)


def _make_seg_scatter(F):
  cparams = None
  if F < 128:
    cparams = pltpu.CompilerParams(use_tc_tiling_on_sc=False)

  @functools.partial(
      pl.kernel,
      compiler_params=cparams,
      out_type=jax.ShapeDtypeStruct((NC, N_PAD, F), jnp.float32),
      mesh=_mesh,
      scratch_types=[
          pltpu.VMEM((N_CHUNK, C), jnp.int32),
          pltpu.VMEM((N_CHUNK, C), jnp.int32),
          pltpu.VMEM((C, F), jnp.float32),
          pltpu.VMEM((C, F), jnp.float32),
          pltpu.VMEM((16, F), jnp.float32),
          pltpu.VMEM_SHARED((N_PAD, F), jnp.float32),
          pltpu.SemaphoreType.DMA,
          pltpu.SemaphoreType.DMA,
      ],
  )
  def k(g_hbm, src_hbm, dst_hbm, out_hbm, src_v, dst_v, rows0, rows1,
        zero_v, acc, gsem0, gsem1):
    cid = lax.axis_index("c")
    sid = lax.axis_index("s")
    wid = sid * NC + cid
    row0 = sid * ROWS_PER_TILE

    pltpu.sync_copy(src_hbm.at[pl.ds(wid * N_CHUNK, N_CHUNK)], src_v)
    pltpu.sync_copy(dst_hbm.at[pl.ds(wid * N_CHUNK, N_CHUNK)], dst_v)

    _zero_fill(zero_v, F)
    @pl.loop(0, ROWS_PER_TILE, step=16)
    def _(r):
      pltpu.sync_copy(zero_v, acc.at[pl.ds(row0 + r, 16)])
    plsc.subcore_barrier()

    def g_start(i, buf, sem):
      pltpu.async_copy(g_hbm.at[src_v.at[i]], buf, sem)

    def g_wait(i, buf, sem):
      pltpu.make_async_copy(g_hbm.at[src_v.at[i]], buf, sem).wait()

    def s_sync(i, buf):
      pltpu.sync_copy(buf, acc.at[dst_v.at[i]], add=True)

    g_start(0, rows0, gsem0)

    @pl.loop(0, N_CHUNK - 2, step=2)
    def _(i):
      g_wait(i, rows0, gsem0)
      g_start(i + 1, rows1, gsem1)
      s_sync(i, rows0)
      g_wait(i + 1, rows1, gsem1)
      g_start(i + 2, rows0, gsem0)
      s_sync(i + 1, rows1)

    g_wait(N_CHUNK - 2, rows0, gsem0)
    g_start(N_CHUNK - 1, rows1, gsem1)
    s_sync(N_CHUNK - 2, rows0)
    g_wait(N_CHUNK - 1, rows1, gsem1)
    s_sync(N_CHUNK - 1, rows1)

    plsc.subcore_barrier()
    pltpu.sync_copy(acc.at[pl.ds(row0, ROWS_PER_TILE)],
                    out_hbm.at[cid].at[pl.ds(row0, ROWS_PER_TILE)])

  return k


_seg_scatter_hid = _make_seg_scatter(HID)
_seg_scatter_16 = _make_seg_scatter(16)


@functools.partial(
    pl.kernel,
    out_type=jax.ShapeDtypeStruct((NC, N_PAD, 16), jnp.float32),
    mesh=_mesh,
    compiler_params=pltpu.CompilerParams(use_tc_tiling_on_sc=False),
    scratch_types=[
        pltpu.VMEM((N_CHUNK, C), jnp.int32),
        pltpu.VMEM((C, 16), jnp.float32),
        pltpu.VMEM((16, 16), jnp.float32),
        pltpu.VMEM_SHARED((N_PAD, 16), jnp.float32),
    ],
)
def _degree_kernel(dst_hbm, out_hbm, dst_v, ones_v, zero_v, acc):
  cid = lax.axis_index("c")
  sid = lax.axis_index("s")
  wid = sid * NC + cid
  row0 = sid * ROWS_PER_TILE

  pltpu.sync_copy(dst_hbm.at[pl.ds(wid * N_CHUNK, N_CHUNK)], dst_v)

  @pl.loop(0, C)
  def _(r):
    ones_v.at[pl.ds(r, 1), pl.ds(0, 16)][...] = jnp.ones((1, 16), jnp.float32)
  _zero_fill(zero_v, 16)
  @pl.loop(0, ROWS_PER_TILE, step=16)
  def _(r):
    pltpu.sync_copy(zero_v, acc.at[pl.ds(row0 + r, 16)])
  plsc.subcore_barrier()

  @pl.loop(0, N_CHUNK)
  def _(i):
    pltpu.sync_copy(ones_v, acc.at[dst_v.at[i]], add=True)

  plsc.subcore_barrier()
  pltpu.sync_copy(acc.at[pl.ds(row0, ROWS_PER_TILE)],
                  out_hbm.at[cid].at[pl.ds(row0, ROWS_PER_TILE)])


ROWS_BLK = 512
GRID = (N_PAD // ROWS_BLK,)

_DOT = dict(dimension_numbers=(((1,), (0,)), ((), ())),
            preferred_element_type=jnp.float32,
            precision=lax.Precision.HIGHEST)


def _selu(x):
  alpha = 1.6732632423543772
  scale = 1.0507009873554805
  return scale * jnp.where(x > 0, x, alpha * (jnp.exp(x) - 1.0))


def _dinv(deg_blk):
  d = deg_blk[0] + deg_blk[1] + 1.0
  return lax.rsqrt(d)[:, :1]


def _mm_body(x_ref, w_ref, o_ref):
  o_ref[...] = lax.dot_general(x_ref[...], w_ref[...], **_DOT)


def _scale_body(deg_ref, xw_ref, o_ref):
  o_ref[...] = _dinv(deg_ref[...]) * xw_ref[...]


def _layer_body(p_ref, g_ref, deg_ref, b_ref, w_ref, o_ref):
  dinv = _dinv(deg_ref[...])
  a = p_ref[0] + p_ref[1] + g_ref[...]
  h = _selu(dinv * a + b_ref[...])
  o_ref[...] = lax.dot_general(h * dinv, w_ref[...], **_DOT)


def _final_body(p_ref, g_ref, deg_ref, b_ref, o_ref):
  dinv = _dinv(deg_ref[...])
  o = dinv * (p_ref[0] + p_ref[1] + g_ref[...]) + b_ref[...]
  col = lax.broadcasted_iota(jnp.int32, o.shape, 1)
  xm = jnp.where(col < D_OUT, o, -1e30)
  m = jnp.max(xm, axis=1, keepdims=True)
  lse = jnp.log(jnp.sum(jnp.exp(xm - m), axis=1, keepdims=True)) + m
  o_ref[...] = o - lse


def _rows_spec(f):
  return pl.BlockSpec((ROWS_BLK, f), lambda i: (i, 0))


def _pair_spec(f):
  return pl.BlockSpec((2, ROWS_BLK, f), lambda i: (0, i, 0))


def _full_spec(shape):
  return pl.BlockSpec(shape, lambda i: tuple(0 for _ in shape))


def _mm_call(x_p, w):
  return pl.pallas_call(
      _mm_body, grid=GRID,
      in_specs=[_rows_spec(x_p.shape[1]), _full_spec(w.shape)],
      out_specs=_rows_spec(w.shape[1]),
      out_shape=jax.ShapeDtypeStruct((N_PAD, w.shape[1]), jnp.float32),
  )(x_p, w)


def _scale_call(degp, xw):
  return pl.pallas_call(
      _scale_body, grid=GRID,
      in_specs=[_pair_spec(16), _rows_spec(HID)],
      out_specs=_rows_spec(HID),
      out_shape=jax.ShapeDtypeStruct((N_PAD, HID), jnp.float32),
  )(degp, xw)


def _layer_call(p, g, degp, b, w):
  f_in, f_out = w.shape
  return pl.pallas_call(
      _layer_body, grid=GRID,
      in_specs=[_pair_spec(f_in), _rows_spec(f_in), _pair_spec(16),
                _full_spec((1, f_in)), _full_spec(w.shape)],
      out_specs=_rows_spec(f_out),
      out_shape=jax.ShapeDtypeStruct((N_PAD, f_out), jnp.float32),
  )(p, g, degp, b, w)


def _final_call(p, g, degp, b):
  return pl.pallas_call(
      _final_body, grid=GRID,
      in_specs=[_pair_spec(16), _rows_spec(16), _pair_spec(16),
                _full_spec((1, 16))],
      out_specs=_rows_spec(16),
      out_shape=jax.ShapeDtypeStruct((N_PAD, 16), jnp.float32),
  )(p, g, degp, b)


def kernel(x, edge_index, W1, b1, W2, b2, W3, b3):
  src = edge_index[0]
  dst = edge_index[1]
  pad_i = jnp.arange(E_PAD - E, dtype=jnp.int32) % PAD_SPREAD
  src2 = jnp.concatenate([src, pad_i]).reshape(E_PAD // C, C)
  dst2 = jnp.concatenate([dst, N + pad_i]).reshape(E_PAD // C, C)
  x_p = jnp.pad(x, ((0, N_PAD - N), (0, 0)))
  w3p = jnp.pad(W3, ((0, 0), (0, 16 - D_OUT)))
  b1r = b1.reshape(1, HID)
  b2r = b2.reshape(1, HID)
  b3r = jnp.pad(b3, (0, 16 - D_OUT)).reshape(1, 16)

  degp = _degree_kernel(dst2)
  xw1 = _mm_call(x_p, W1)
  g1 = _scale_call(degp, xw1)
  p1 = _seg_scatter_hid(g1, src2, dst2)
  g2 = _layer_call(p1, g1, degp, b1r, W2)
  p2 = _seg_scatter_hid(g2, src2, dst2)
  g3 = _layer_call(p2, g2, degp, b2r, w3p)
  p3 = _seg_scatter_16(g3, src2, dst2)
  o = _final_call(p3, g3, degp, b3r)
  return o[:N, :D_OUT]


# speedup vs baseline: 19.0558x; 1.0990x over previous
import functools

import jax
import jax.numpy as jnp
from jax import lax
from jax.experimental import pallas as pl
from jax.experimental.pallas import tpu as pltpu
from jax.experimental.pallas import tpu_sc as plsc

N = 10000
E = 160000
D_IN = 256
HID = 128
D_OUT = 12

NC, NS = 2, 16
NW = NC * NS
N_PAD = 10240
E_PAD = 163840
PER_W = E_PAD // NW
ROWS_PER_TILE = N_PAD // NS
PAD_SPREAD = 240

_mesh = plsc.VectorSubcoreMesh(core_axis_name="c", subcore_axis_name="s",
                               num_cores=NC, num_subcores=NS)


def _zero_fill(zero_v, F):
  @pl.loop(0, 16)
  def _(r):
    @pl.loop(0, F, step=16)
    def _(j):
      zero_v.at[pl.ds(r, 1), pl.ds(j, 16)][...] = jnp.zeros((1, 16), jnp.float32)


def _make_seg_scatter(F, C, nbuf):
  n_chunk = PER_W // C
  cparams = None
  if F < 128:
    cparams = pltpu.CompilerParams(use_tc_tiling_on_sc=False)

  @functools.partial(
      pl.kernel,
      compiler_params=cparams,
      out_type=jax.ShapeDtypeStruct((NC, N_PAD, F), jnp.float32),
      mesh=_mesh,
      scratch_types=[
          pltpu.VMEM((n_chunk, C), jnp.int32),
          pltpu.VMEM((n_chunk, C), jnp.int32),
          pltpu.VMEM((nbuf, C, F), jnp.float32),
          pltpu.VMEM((16, F), jnp.float32),
          pltpu.VMEM_SHARED((N_PAD, F), jnp.float32),
          [pltpu.SemaphoreType.DMA] * nbuf,
          [pltpu.SemaphoreType.DMA] * nbuf,
      ],
  )
  def k(g_hbm, src_hbm, dst_hbm, out_hbm, src_v, dst_v, rows_v,
        zero_v, acc, gsems, ssems):
    cid = lax.axis_index("c")
    sid = lax.axis_index("s")
    wid = sid * NC + cid
    row0 = sid * ROWS_PER_TILE

    pltpu.sync_copy(src_hbm.at[pl.ds(wid * n_chunk, n_chunk)], src_v)
    pltpu.sync_copy(dst_hbm.at[pl.ds(wid * n_chunk, n_chunk)], dst_v)

    _zero_fill(zero_v, F)
    @pl.loop(0, ROWS_PER_TILE, step=16)
    def _(r):
      pltpu.sync_copy(zero_v, acc.at[pl.ds(row0 + r, 16)])
    plsc.subcore_barrier()

    bufs = [rows_v.at[b] for b in range(nbuf)]

    def g_start(i, b):
      pltpu.async_copy(g_hbm.at[src_v.at[i]], bufs[b], gsems[b])

    def g_wait(i, b):
      pltpu.make_async_copy(g_hbm.at[src_v.at[i]], bufs[b], gsems[b]).wait()

    def s_start(i, b):
      pltpu.async_copy(bufs[b], acc.at[dst_v.at[i]], ssems[b], add=True)

    def s_wait(i, b):
      pltpu.make_async_copy(bufs[b], acc.at[dst_v.at[i]], ssems[b]).wait()

    if nbuf == 4:
      g_start(0, 0)
      g_start(1, 1)
      g_wait(0, 0); s_start(0, 0); g_start(2, 2)
      g_wait(1, 1); s_start(1, 1); g_start(3, 3)
      g_wait(2, 2); s_start(2, 2); s_wait(0, 0); g_start(4, 0)
      g_wait(3, 3); s_start(3, 3); s_wait(1, 1); g_start(5, 1)

      @pl.loop(4, n_chunk - 4, step=4)
      def _(i0):
        for j in range(4):
          i = i0 + j
          b, bn = j, (j + 2) % 4
          g_wait(i, b)
          s_start(i, b)
          s_wait(i - 2, bn)
          g_start(i + 2, bn)

      e = n_chunk - 4
      g_wait(e, 0); s_start(e, 0); s_wait(e - 2, 2); g_start(e + 2, 2)
      g_wait(e + 1, 1); s_start(e + 1, 1); s_wait(e - 1, 3); g_start(e + 3, 3)
      g_wait(e + 2, 2); s_start(e + 2, 2); s_wait(e, 0)
      g_wait(e + 3, 3); s_start(e + 3, 3); s_wait(e + 1, 1)
      s_wait(e + 2, 2)
      s_wait(e + 3, 3)
    else:
      g_start(0, 0)
      g_wait(0, 0); s_start(0, 0); g_start(1, 1)

      @pl.loop(1, n_chunk - 1, step=2)
      def _(i):
        g_wait(i, 1); s_start(i, 1); s_wait(i - 1, 0); g_start(i + 1, 0)
        g_wait(i + 1, 0); s_start(i + 1, 0); s_wait(i, 1); g_start(i + 2, 1)

      e = n_chunk - 1
      g_wait(e, 1); s_start(e, 1); s_wait(e - 1, 0); s_wait(e, 1)

    plsc.subcore_barrier()
    pltpu.sync_copy(acc.at[pl.ds(row0, ROWS_PER_TILE)],
                    out_hbm.at[cid].at[pl.ds(row0, ROWS_PER_TILE)])

  return k


C_E = 128
_seg_scatter_hid = _make_seg_scatter(HID, C_E, 2)
_seg_scatter_16 = _make_seg_scatter(16, C_E, 4)


@functools.partial(
    pl.kernel,
    out_type=jax.ShapeDtypeStruct((NC, N_PAD, 16), jnp.float32),
    mesh=_mesh,
    compiler_params=pltpu.CompilerParams(use_tc_tiling_on_sc=False),
    scratch_types=[
        pltpu.VMEM((PER_W // C_E, C_E), jnp.int32),
        pltpu.VMEM((C_E, 16), jnp.float32),
        pltpu.VMEM((16, 16), jnp.float32),
        pltpu.VMEM_SHARED((N_PAD, 16), jnp.float32),
        pltpu.SemaphoreType.DMA,
    ],
)
def _degree_kernel(dst_hbm, out_hbm, dst_v, ones_v, zero_v, acc, sem):
  n_chunk = PER_W // C_E
  cid = lax.axis_index("c")
  sid = lax.axis_index("s")
  wid = sid * NC + cid
  row0 = sid * ROWS_PER_TILE

  pltpu.sync_copy(dst_hbm.at[pl.ds(wid * n_chunk, n_chunk)], dst_v)

  @pl.loop(0, C_E)
  def _(r):
    ones_v.at[pl.ds(r, 1), pl.ds(0, 16)][...] = jnp.ones((1, 16), jnp.float32)
  _zero_fill(zero_v, 16)
  @pl.loop(0, ROWS_PER_TILE, step=16)
  def _(r):
    pltpu.sync_copy(zero_v, acc.at[pl.ds(row0 + r, 16)])
  plsc.subcore_barrier()

  @pl.loop(0, 8)
  def _(i):
    pltpu.async_copy(ones_v, acc.at[dst_v.at[i]], sem, add=True)

  @pl.loop(8, n_chunk)
  def _(i):
    pltpu.async_copy(ones_v, acc.at[dst_v.at[i]], sem, add=True)
    pltpu.make_async_copy(ones_v, acc.at[dst_v.at[i - 8]], sem).wait()

  @pl.loop(n_chunk - 8, n_chunk)
  def _(i):
    pltpu.make_async_copy(ones_v, acc.at[dst_v.at[i]], sem).wait()

  plsc.subcore_barrier()
  pltpu.sync_copy(acc.at[pl.ds(row0, ROWS_PER_TILE)],
                  out_hbm.at[cid].at[pl.ds(row0, ROWS_PER_TILE)])


ROWS_BLK = 512
GRID = (N_PAD // ROWS_BLK,)

_DOT = dict(dimension_numbers=(((1,), (0,)), ((), ())),
            preferred_element_type=jnp.float32,
            precision=lax.Precision.DEFAULT)


def _selu(x):
  alpha = 1.6732632423543772
  scale = 1.0507009873554805
  return scale * jnp.where(x > 0, x, alpha * (jnp.exp(x) - 1.0))


def _dinv(deg_blk):
  d = deg_blk[0] + deg_blk[1] + 1.0
  return lax.rsqrt(d)[:, :1]


def _mm_scale_body(deg_ref, x_ref, w_ref, o_ref):
  o_ref[...] = _dinv(deg_ref[...]) * lax.dot_general(x_ref[...], w_ref[...],
                                                     **_DOT)


def _layer_body(p_ref, g_ref, deg_ref, b_ref, w_ref, o_ref):
  dinv = _dinv(deg_ref[...])
  a = p_ref[0] + p_ref[1] + g_ref[...]
  h = _selu(dinv * a + b_ref[...])
  o_ref[...] = lax.dot_general(h * dinv, w_ref[...], **_DOT)


def _final_body(p_ref, g_ref, deg_ref, b_ref, o_ref):
  dinv = _dinv(deg_ref[...])
  o = dinv * (p_ref[0] + p_ref[1] + g_ref[...]) + b_ref[...]
  col = lax.broadcasted_iota(jnp.int32, o.shape, 1)
  xm = jnp.where(col < D_OUT, o, -1e30)
  m = jnp.max(xm, axis=1, keepdims=True)
  lse = jnp.log(jnp.sum(jnp.exp(xm - m), axis=1, keepdims=True)) + m
  o_ref[...] = o - lse


def _rows_spec(f):
  return pl.BlockSpec((ROWS_BLK, f), lambda i: (i, 0))


def _pair_spec(f):
  return pl.BlockSpec((2, ROWS_BLK, f), lambda i: (0, i, 0))


def _full_spec(shape):
  return pl.BlockSpec(shape, lambda i: tuple(0 for _ in shape))


def _mm_scale_call(degp, x_p, w):
  return pl.pallas_call(
      _mm_scale_body, grid=GRID,
      in_specs=[_pair_spec(16), _rows_spec(x_p.shape[1]), _full_spec(w.shape)],
      out_specs=_rows_spec(w.shape[1]),
      out_shape=jax.ShapeDtypeStruct((N_PAD, w.shape[1]), jnp.float32),
  )(degp, x_p, w)


def _layer_call(p, g, degp, b, w):
  f_in, f_out = w.shape
  return pl.pallas_call(
      _layer_body, grid=GRID,
      in_specs=[_pair_spec(f_in), _rows_spec(f_in), _pair_spec(16),
                _full_spec((1, f_in)), _full_spec(w.shape)],
      out_specs=_rows_spec(f_out),
      out_shape=jax.ShapeDtypeStruct((N_PAD, f_out), jnp.float32),
  )(p, g, degp, b, w)


def _final_call(p, g, degp, b):
  return pl.pallas_call(
      _final_body, grid=GRID,
      in_specs=[_pair_spec(16), _rows_spec(16), _pair_spec(16),
                _full_spec((1, 16))],
      out_specs=_rows_spec(16),
      out_shape=jax.ShapeDtypeStruct((N_PAD, 16), jnp.float32),
  )(p, g, degp, b)


def kernel(x, edge_index, W1, b1, W2, b2, W3, b3):
  src = edge_index[0]
  dst = edge_index[1]
  pad_i = jnp.arange(E_PAD - E, dtype=jnp.int32) % PAD_SPREAD
  src2 = jnp.concatenate([src, pad_i]).reshape(E_PAD // C_E, C_E)
  dst2 = jnp.concatenate([dst, N + pad_i]).reshape(E_PAD // C_E, C_E)
  x_p = jnp.pad(x, ((0, N_PAD - N), (0, 0)))
  w3p = jnp.pad(W3, ((0, 0), (0, 16 - D_OUT)))
  b1r = b1.reshape(1, HID)
  b2r = b2.reshape(1, HID)
  b3r = jnp.pad(b3, (0, 16 - D_OUT)).reshape(1, 16)

  degp = _degree_kernel(dst2)
  g1 = _mm_scale_call(degp, x_p, W1)
  p1 = _seg_scatter_hid(g1, src2, dst2)
  g2 = _layer_call(p1, g1, degp, b1r, W2)
  p2 = _seg_scatter_hid(g2, src2, dst2)
  g3 = _layer_call(p2, g2, degp, b2r, w3p)
  p3 = _seg_scatter_16(g3, src2, dst2)
  o = _final_call(p3, g3, degp, b3r)
  return o[:N, :D_OUT]


# speedup vs baseline: 19.9871x; 1.0489x over previous
import functools

import jax
import jax.numpy as jnp
from jax import lax
from jax.experimental import pallas as pl
from jax.experimental.pallas import tpu as pltpu
from jax.experimental.pallas import tpu_sc as plsc

N = 10000
E = 160000
D_IN = 256
HID = 128
D_OUT = 12

NC, NS = 2, 16
NW = NC * NS
N_PAD = 10240
E_PAD = 163840
PER_W = E_PAD // NW
ROWS_PER_TILE = N_PAD // NS
PAD_SPREAD = 240

_mesh = plsc.VectorSubcoreMesh(core_axis_name="c", subcore_axis_name="s",
                               num_cores=NC, num_subcores=NS)


def _zero_fill(zero_v, F):
  @pl.loop(0, 16)
  def _(r):
    @pl.loop(0, F, step=16)
    def _(j):
      zero_v.at[pl.ds(r, 1), pl.ds(j, 16)][...] = jnp.zeros((1, 16), jnp.float32)


def _make_seg_scatter(F, C, nbuf):
  n_chunk = PER_W // C
  cparams = pltpu.CompilerParams(use_tc_tiling_on_sc=False)

  @functools.partial(
      pl.kernel,
      compiler_params=cparams,
      out_type=jax.ShapeDtypeStruct((NC, N_PAD, F), jnp.float32),
      mesh=_mesh,
      scratch_types=[
          pltpu.VMEM((n_chunk, C), jnp.int32),
          pltpu.VMEM((n_chunk, C), jnp.int32),
          pltpu.VMEM((nbuf, C, F), jnp.float32),
          pltpu.VMEM((16, F), jnp.float32),
          pltpu.VMEM_SHARED((N_PAD, F), jnp.float32),
          [pltpu.SemaphoreType.DMA] * nbuf,
          [pltpu.SemaphoreType.DMA] * nbuf,
      ],
  )
  def k(g_hbm, src_hbm, dst_hbm, out_hbm, src_v, dst_v, rows_v,
        zero_v, acc, gsems, ssems):
    cid = lax.axis_index("c")
    sid = lax.axis_index("s")
    wid = sid * NC + cid
    row0 = sid * ROWS_PER_TILE

    pltpu.sync_copy(src_hbm.at[pl.ds(wid * n_chunk, n_chunk)], src_v)
    pltpu.sync_copy(dst_hbm.at[pl.ds(wid * n_chunk, n_chunk)], dst_v)

    _zero_fill(zero_v, F)
    @pl.loop(0, ROWS_PER_TILE, step=16)
    def _(r):
      pltpu.sync_copy(zero_v, acc.at[pl.ds(row0 + r, 16)])
    plsc.subcore_barrier()

    bufs = [rows_v.at[b] for b in range(nbuf)]

    def g_start(i, b):
      pltpu.async_copy(g_hbm.at[src_v.at[i]], bufs[b], gsems[b])

    def g_wait(i, b):
      pltpu.make_async_copy(g_hbm.at[src_v.at[i]], bufs[b], gsems[b]).wait()

    def s_start(i, b):
      pltpu.async_copy(bufs[b], acc.at[dst_v.at[i]], ssems[b], add=True)

    def s_wait(i, b):
      pltpu.make_async_copy(bufs[b], acc.at[dst_v.at[i]], ssems[b]).wait()

    lag = nbuf // 2

    def slot(i, b, static):
      g_wait(i, b)
      s_start(i, b)
      if not static or i - lag >= 0:
        s_wait(i - lag, (b - lag) % nbuf)
      if not static or i + lag <= n_chunk - 1:
        g_start(i + lag, (b + lag) % nbuf)

    loop_end = ((n_chunk - lag) // nbuf) * nbuf
    for i in range(min(lag, n_chunk)):
      g_start(i, i % nbuf)
    for i in range(min(nbuf, loop_end)):
      slot(i, i % nbuf, static=True)

    @pl.loop(nbuf, loop_end, step=nbuf)
    def _(i0):
      for j in range(nbuf):
        slot(i0 + j, j, static=False)

    for i in range(loop_end, n_chunk):
      slot(i, i % nbuf, static=True)
    for i in range(max(0, n_chunk - lag), n_chunk):
      s_wait(i, i % nbuf)

    plsc.subcore_barrier()
    pltpu.sync_copy(acc.at[pl.ds(row0, ROWS_PER_TILE)],
                    out_hbm.at[cid].at[pl.ds(row0, ROWS_PER_TILE)])

  return k


C_E = 128
C_HID = 64
_seg_scatter_hid = _make_seg_scatter(HID, C_HID, 4)
_seg_scatter_16 = _make_seg_scatter(16, C_E, 8)


@functools.partial(
    pl.kernel,
    out_type=jax.ShapeDtypeStruct((NC, N_PAD, 16), jnp.float32),
    mesh=_mesh,
    compiler_params=pltpu.CompilerParams(use_tc_tiling_on_sc=False),
    scratch_types=[
        pltpu.VMEM((PER_W // C_E, C_E), jnp.int32),
        pltpu.VMEM((C_E, 16), jnp.float32),
        pltpu.VMEM((16, 16), jnp.float32),
        pltpu.VMEM_SHARED((N_PAD, 16), jnp.float32),
        pltpu.SemaphoreType.DMA,
    ],
)
def _degree_kernel(dst_hbm, out_hbm, dst_v, ones_v, zero_v, acc, sem):
  n_chunk = PER_W // C_E
  cid = lax.axis_index("c")
  sid = lax.axis_index("s")
  wid = sid * NC + cid
  row0 = sid * ROWS_PER_TILE

  pltpu.sync_copy(dst_hbm.at[pl.ds(wid * n_chunk, n_chunk)], dst_v)

  @pl.loop(0, C_E)
  def _(r):
    ones_v.at[pl.ds(r, 1), pl.ds(0, 16)][...] = jnp.ones((1, 16), jnp.float32)
  _zero_fill(zero_v, 16)
  @pl.loop(0, ROWS_PER_TILE, step=16)
  def _(r):
    pltpu.sync_copy(zero_v, acc.at[pl.ds(row0 + r, 16)])
  plsc.subcore_barrier()

  @pl.loop(0, 8)
  def _(i):
    pltpu.async_copy(ones_v, acc.at[dst_v.at[i]], sem, add=True)

  @pl.loop(8, n_chunk)
  def _(i):
    pltpu.async_copy(ones_v, acc.at[dst_v.at[i]], sem, add=True)
    pltpu.make_async_copy(ones_v, acc.at[dst_v.at[i - 8]], sem).wait()

  @pl.loop(n_chunk - 8, n_chunk)
  def _(i):
    pltpu.make_async_copy(ones_v, acc.at[dst_v.at[i]], sem).wait()

  plsc.subcore_barrier()
  pltpu.sync_copy(acc.at[pl.ds(row0, ROWS_PER_TILE)],
                  out_hbm.at[cid].at[pl.ds(row0, ROWS_PER_TILE)])


ROWS_BLK = 512
GRID = (N_PAD // ROWS_BLK,)

_DOT = dict(dimension_numbers=(((1,), (0,)), ((), ())),
            preferred_element_type=jnp.float32,
            precision=lax.Precision.DEFAULT)


def _selu(x):
  alpha = 1.6732632423543772
  scale = 1.0507009873554805
  return scale * jnp.where(x > 0, x, alpha * (jnp.exp(x) - 1.0))


def _dinv(deg_blk):
  d = deg_blk[0] + deg_blk[1] + 1.0
  return lax.rsqrt(d)[:, :1]


def _mm_scale_body(deg_ref, x_ref, w_ref, o_ref):
  o_ref[...] = _dinv(deg_ref[...]) * lax.dot_general(x_ref[...], w_ref[...],
                                                     **_DOT)


def _layer_body(p_ref, g_ref, deg_ref, b_ref, w_ref, o_ref):
  dinv = _dinv(deg_ref[...])
  a = p_ref[0] + p_ref[1] + g_ref[...]
  h = _selu(dinv * a + b_ref[...])
  o_ref[...] = lax.dot_general(h * dinv, w_ref[...], **_DOT)


def _final_body(p_ref, g_ref, deg_ref, b_ref, o_ref):
  dinv = _dinv(deg_ref[...])
  o = dinv * (p_ref[0] + p_ref[1] + g_ref[...]) + b_ref[...]
  col = lax.broadcasted_iota(jnp.int32, o.shape, 1)
  xm = jnp.where(col < D_OUT, o, -1e30)
  m = jnp.max(xm, axis=1, keepdims=True)
  lse = jnp.log(jnp.sum(jnp.exp(xm - m), axis=1, keepdims=True)) + m
  o_ref[...] = o - lse


def _rows_spec(f):
  return pl.BlockSpec((ROWS_BLK, f), lambda i: (i, 0))


def _pair_spec(f):
  return pl.BlockSpec((2, ROWS_BLK, f), lambda i: (0, i, 0))


def _full_spec(shape):
  return pl.BlockSpec(shape, lambda i: tuple(0 for _ in shape))


def _mm_scale_call(degp, x_p, w):
  return pl.pallas_call(
      _mm_scale_body, grid=GRID,
      in_specs=[_pair_spec(16), _rows_spec(x_p.shape[1]), _full_spec(w.shape)],
      out_specs=_rows_spec(w.shape[1]),
      out_shape=jax.ShapeDtypeStruct((N_PAD, w.shape[1]), jnp.float32),
  )(degp, x_p, w)


def _layer_call(p, g, degp, b, w):
  f_in, f_out = w.shape
  return pl.pallas_call(
      _layer_body, grid=GRID,
      in_specs=[_pair_spec(f_in), _rows_spec(f_in), _pair_spec(16),
                _full_spec((1, f_in)), _full_spec(w.shape)],
      out_specs=_rows_spec(f_out),
      out_shape=jax.ShapeDtypeStruct((N_PAD, f_out), jnp.float32),
  )(p, g, degp, b, w)


def _final_call(p, g, degp, b):
  return pl.pallas_call(
      _final_body, grid=GRID,
      in_specs=[_pair_spec(16), _rows_spec(16), _pair_spec(16),
                _full_spec((1, 16))],
      out_specs=_rows_spec(16),
      out_shape=jax.ShapeDtypeStruct((N_PAD, 16), jnp.float32),
  )(p, g, degp, b)


def kernel(x, edge_index, W1, b1, W2, b2, W3, b3):
  src = edge_index[0]
  dst = edge_index[1]
  pad_i = jnp.arange(E_PAD - E, dtype=jnp.int32) % PAD_SPREAD
  src_p = jnp.concatenate([src, pad_i])
  dst_p = jnp.concatenate([dst, N + pad_i])
  src2 = src_p.reshape(E_PAD // C_E, C_E)
  dst2 = dst_p.reshape(E_PAD // C_E, C_E)
  src2h = src_p.reshape(E_PAD // C_HID, C_HID)
  dst2h = dst_p.reshape(E_PAD // C_HID, C_HID)
  x_p = jnp.pad(x, ((0, N_PAD - N), (0, 0)))
  w3p = jnp.pad(W3, ((0, 0), (0, 16 - D_OUT)))
  b1r = b1.reshape(1, HID)
  b2r = b2.reshape(1, HID)
  b3r = jnp.pad(b3, (0, 16 - D_OUT)).reshape(1, 16)

  degp = _degree_kernel(dst2)
  g1 = _mm_scale_call(degp, x_p, W1)
  p1 = _seg_scatter_hid(g1, src2h, dst2h)
  g2 = _layer_call(p1, g1, degp, b1r, W2)
  p2 = _seg_scatter_hid(g2, src2h, dst2h)
  g3 = _layer_call(p2, g2, degp, b2r, w3p)
  p3 = _seg_scatter_16(g3, src2, dst2)
  o = _final_call(p3, g3, degp, b3r)
  return o[:N, :D_OUT]


# speedup vs baseline: 21.9971x; 1.1006x over previous
import functools

import jax
import jax.numpy as jnp
from jax import lax
from jax.experimental import pallas as pl
from jax.experimental.pallas import tpu as pltpu
from jax.experimental.pallas import tpu_sc as plsc

N = 10000
E = 160000
D_IN = 256
HID = 128
D_OUT = 12

NC, NS = 2, 16
NW = NC * NS
N_PAD = 10240
E_PAD = 163840
PER_W = E_PAD // NW
ROWS_PER_TILE = N_PAD // NS
OUT_PER_TILE = N // NS
PAD_SPREAD = 240

_mesh = plsc.VectorSubcoreMesh(core_axis_name="c", subcore_axis_name="s",
                               num_cores=NC, num_subcores=NS)


def _zero_fill(zero_v, F):
  @pl.loop(0, 16)
  def _(r):
    @pl.loop(0, F, step=16)
    def _(j):
      zero_v.at[pl.ds(r, 1), pl.ds(j, 16)][...] = jnp.zeros((1, 16), jnp.float32)


def _make_seg_scatter(F, C, nbuf):
  n_chunk = PER_W // C
  cparams = pltpu.CompilerParams(use_tc_tiling_on_sc=False)

  @functools.partial(
      pl.kernel,
      compiler_params=cparams,
      out_type=jax.ShapeDtypeStruct((NC, N, F), jnp.float32),
      mesh=_mesh,
      scratch_types=[
          pltpu.VMEM((n_chunk, C), jnp.int32),
          pltpu.VMEM((n_chunk, C), jnp.int32),
          pltpu.VMEM((nbuf, C, F), jnp.float32),
          pltpu.VMEM((16, F), jnp.float32),
          pltpu.VMEM_SHARED((N_PAD, F), jnp.float32),
          [pltpu.SemaphoreType.DMA] * nbuf,
          [pltpu.SemaphoreType.DMA] * nbuf,
      ],
  )
  def k(g_hbm, src_hbm, dst_hbm, out_hbm, src_v, dst_v, rows_v,
        zero_v, acc, gsems, ssems):
    cid = lax.axis_index("c")
    sid = lax.axis_index("s")
    wid = sid * NC + cid
    row0 = sid * ROWS_PER_TILE

    pltpu.sync_copy(src_hbm.at[pl.ds(wid * n_chunk, n_chunk)], src_v)
    pltpu.sync_copy(dst_hbm.at[pl.ds(wid * n_chunk, n_chunk)], dst_v)

    _zero_fill(zero_v, F)
    @pl.loop(0, ROWS_PER_TILE, step=16)
    def _(r):
      pltpu.sync_copy(zero_v, acc.at[pl.ds(row0 + r, 16)])
    plsc.subcore_barrier()

    bufs = [rows_v.at[b] for b in range(nbuf)]

    def g_start(i, b):
      pltpu.async_copy(g_hbm.at[src_v.at[i]], bufs[b], gsems[b])

    def g_wait(i, b):
      pltpu.make_async_copy(g_hbm.at[src_v.at[i]], bufs[b], gsems[b]).wait()

    def s_start(i, b):
      pltpu.async_copy(bufs[b], acc.at[dst_v.at[i]], ssems[b], add=True)

    def s_wait(i, b):
      pltpu.make_async_copy(bufs[b], acc.at[dst_v.at[i]], ssems[b]).wait()

    lag = nbuf // 2

    def slot(i, b, static):
      g_wait(i, b)
      s_start(i, b)
      if not static or i - lag >= 0:
        s_wait(i - lag, (b - lag) % nbuf)
      if not static or i + lag <= n_chunk - 1:
        g_start(i + lag, (b + lag) % nbuf)

    loop_end = ((n_chunk - lag) // nbuf) * nbuf
    for i in range(min(lag, n_chunk)):
      g_start(i, i % nbuf)
    for i in range(min(nbuf, loop_end)):
      slot(i, i % nbuf, static=True)

    @pl.loop(nbuf, loop_end, step=nbuf)
    def _(i0):
      for j in range(nbuf):
        slot(i0 + j, j, static=False)

    for i in range(loop_end, n_chunk):
      slot(i, i % nbuf, static=True)
    for i in range(max(0, n_chunk - lag), n_chunk):
      s_wait(i, i % nbuf)

    plsc.subcore_barrier()
    o0 = sid * OUT_PER_TILE
    pltpu.sync_copy(acc.at[pl.ds(o0, OUT_PER_TILE)],
                    out_hbm.at[cid].at[pl.ds(o0, OUT_PER_TILE)])

  return k


C_E = 128
C_HID = 64
_seg_scatter_hid = _make_seg_scatter(HID, C_HID, 4)
_seg_scatter_16 = _make_seg_scatter(16, C_E, 8)


@functools.partial(
    pl.kernel,
    out_type=jax.ShapeDtypeStruct((NC, N, 16), jnp.float32),
    mesh=_mesh,
    compiler_params=pltpu.CompilerParams(use_tc_tiling_on_sc=False),
    scratch_types=[
        pltpu.VMEM((PER_W // C_E, C_E), jnp.int32),
        pltpu.VMEM((C_E, 16), jnp.float32),
        pltpu.VMEM((16, 16), jnp.float32),
        pltpu.VMEM_SHARED((N_PAD, 16), jnp.float32),
        pltpu.SemaphoreType.DMA,
    ],
)
def _degree_kernel(dst_hbm, out_hbm, dst_v, ones_v, zero_v, acc, sem):
  n_chunk = PER_W // C_E
  cid = lax.axis_index("c")
  sid = lax.axis_index("s")
  wid = sid * NC + cid
  row0 = sid * ROWS_PER_TILE

  pltpu.sync_copy(dst_hbm.at[pl.ds(wid * n_chunk, n_chunk)], dst_v)

  @pl.loop(0, C_E)
  def _(r):
    ones_v.at[pl.ds(r, 1), pl.ds(0, 16)][...] = jnp.ones((1, 16), jnp.float32)
  _zero_fill(zero_v, 16)
  @pl.loop(0, ROWS_PER_TILE, step=16)
  def _(r):
    pltpu.sync_copy(zero_v, acc.at[pl.ds(row0 + r, 16)])
  plsc.subcore_barrier()

  @pl.loop(0, 8)
  def _(i):
    pltpu.async_copy(ones_v, acc.at[dst_v.at[i]], sem, add=True)

  @pl.loop(8, n_chunk)
  def _(i):
    pltpu.async_copy(ones_v, acc.at[dst_v.at[i]], sem, add=True)
    pltpu.make_async_copy(ones_v, acc.at[dst_v.at[i - 8]], sem).wait()

  @pl.loop(n_chunk - 8, n_chunk)
  def _(i):
    pltpu.make_async_copy(ones_v, acc.at[dst_v.at[i]], sem).wait()

  plsc.subcore_barrier()
  o0 = sid * OUT_PER_TILE
  pltpu.sync_copy(acc.at[pl.ds(o0, OUT_PER_TILE)],
                  out_hbm.at[cid].at[pl.ds(o0, OUT_PER_TILE)])


ROWS_BLK = 1000
GRID = (N // ROWS_BLK,)

_DOT = dict(dimension_numbers=(((1,), (0,)), ((), ())),
            preferred_element_type=jnp.float32,
            precision=lax.Precision.DEFAULT)


def _selu(x):
  alpha = 1.6732632423543772
  scale = 1.0507009873554805
  return scale * jnp.where(x > 0, x, alpha * (jnp.exp(x) - 1.0))


def _dinv(deg_blk):
  d = deg_blk[0] + deg_blk[1] + 1.0
  return lax.rsqrt(d)[:, :1]


def _mm_scale_body(deg_ref, x_ref, w_ref, o_ref):
  o_ref[...] = _dinv(deg_ref[...]) * lax.dot_general(x_ref[...], w_ref[...],
                                                     **_DOT)


def _layer_body(p_ref, g_ref, deg_ref, b_ref, w_ref, o_ref):
  dinv = _dinv(deg_ref[...])
  a = p_ref[0] + p_ref[1] + g_ref[...]
  h = _selu(dinv * a + b_ref[...])
  o_ref[...] = lax.dot_general(h * dinv, w_ref[...], **_DOT)


def _final_body(p_ref, g_ref, deg_ref, b_ref, o_ref):
  dinv = _dinv(deg_ref[...])
  o = dinv * (p_ref[0] + p_ref[1] + g_ref[...]) + b_ref[...]
  col = lax.broadcasted_iota(jnp.int32, o.shape, 1)
  xm = jnp.where(col < D_OUT, o, -1e30)
  m = jnp.max(xm, axis=1, keepdims=True)
  lse = jnp.log(jnp.sum(jnp.exp(xm - m), axis=1, keepdims=True)) + m
  o_ref[...] = (o - lse)[:, :D_OUT]


def _rows_spec(f):
  return pl.BlockSpec((ROWS_BLK, f), lambda i: (i, 0))


def _pair_spec(f):
  return pl.BlockSpec((2, ROWS_BLK, f), lambda i: (0, i, 0))


def _full_spec(shape):
  return pl.BlockSpec(shape, lambda i: tuple(0 for _ in shape))


def _mm_scale_call(degp, x, w):
  return pl.pallas_call(
      _mm_scale_body, grid=GRID,
      in_specs=[_pair_spec(16), _rows_spec(x.shape[1]), _full_spec(w.shape)],
      out_specs=_rows_spec(w.shape[1]),
      out_shape=jax.ShapeDtypeStruct((N, w.shape[1]), jnp.float32),
  )(degp, x, w)


def _layer_call(p, g, degp, b, w):
  f_in, f_out = w.shape
  return pl.pallas_call(
      _layer_body, grid=GRID,
      in_specs=[_pair_spec(f_in), _rows_spec(f_in), _pair_spec(16),
                _full_spec((1, f_in)), _full_spec(w.shape)],
      out_specs=_rows_spec(f_out),
      out_shape=jax.ShapeDtypeStruct((N, f_out), jnp.float32),
  )(p, g, degp, b, w)


def _final_call(p, g, degp, b):
  return pl.pallas_call(
      _final_body, grid=GRID,
      in_specs=[_pair_spec(16), _rows_spec(16), _pair_spec(16),
                _full_spec((1, 16))],
      out_specs=_rows_spec(D_OUT),
      out_shape=jax.ShapeDtypeStruct((N, D_OUT), jnp.float32),
  )(p, g, degp, b)


def kernel(x, edge_index, W1, b1, W2, b2, W3, b3):
  src = edge_index[0]
  dst = edge_index[1]
  pad_i = jnp.arange(E_PAD - E, dtype=jnp.int32) % PAD_SPREAD
  src_p = jnp.concatenate([src, pad_i])
  dst_p = jnp.concatenate([dst, N + pad_i])
  src2 = src_p.reshape(E_PAD // C_E, C_E)
  dst2 = dst_p.reshape(E_PAD // C_E, C_E)
  src2h = src_p.reshape(E_PAD // C_HID, C_HID)
  dst2h = dst_p.reshape(E_PAD // C_HID, C_HID)
  w3p = jnp.pad(W3, ((0, 0), (0, 16 - D_OUT)))
  b1r = b1.reshape(1, HID)
  b2r = b2.reshape(1, HID)
  b3r = jnp.pad(b3, (0, 16 - D_OUT)).reshape(1, 16)

  degp = _degree_kernel(dst2)
  g1 = _mm_scale_call(degp, x, W1)
  p1 = _seg_scatter_hid(g1, src2h, dst2h)
  g2 = _layer_call(p1, g1, degp, b1r, W2)
  p2 = _seg_scatter_hid(g2, src2h, dst2h)
  g3 = _layer_call(p2, g2, degp, b2r, w3p)
  p3 = _seg_scatter_16(g3, src2, dst2)
  return _final_call(p3, g3, degp, b3r)


# speedup vs baseline: 22.2424x; 1.0112x over previous
import functools

import jax
import jax.numpy as jnp
from jax import lax
from jax.experimental import pallas as pl
from jax.experimental.pallas import tpu as pltpu
from jax.experimental.pallas import tpu_sc as plsc

N = 10000
E = 160000
D_IN = 256
HID = 128
D_OUT = 12

NC, NS = 2, 16
NW = NC * NS
N_PAD = 10240
E_PAD = 163840
PER_W = E_PAD // NW
ROWS_PER_TILE = N_PAD // NS
OUT_PER_TILE = N // NS
PAD_SPREAD = 240

_mesh = plsc.VectorSubcoreMesh(core_axis_name="c", subcore_axis_name="s",
                               num_cores=NC, num_subcores=NS)


def _zero_fill(zero_v, F):
  @pl.loop(0, 16)
  def _(r):
    @pl.loop(0, F, step=16)
    def _(j):
      zero_v.at[pl.ds(r, 1), pl.ds(j, 16)][...] = jnp.zeros((1, 16), jnp.float32)


def _make_seg_scatter(F, C, nbuf):
  n_chunk = PER_W // C
  cparams = pltpu.CompilerParams(use_tc_tiling_on_sc=False)

  @functools.partial(
      pl.kernel,
      compiler_params=cparams,
      out_type=jax.ShapeDtypeStruct((NC, N, F), jnp.float32),
      mesh=_mesh,
      scratch_types=[
          pltpu.VMEM((n_chunk, C), jnp.int32),
          pltpu.VMEM((n_chunk, C), jnp.int32),
          pltpu.VMEM((nbuf, C, F), jnp.float32),
          pltpu.VMEM((16, F), jnp.float32),
          pltpu.VMEM_SHARED((N_PAD, F), jnp.float32),
          [pltpu.SemaphoreType.DMA] * nbuf,
          [pltpu.SemaphoreType.DMA] * nbuf,
      ],
  )
  def k(g_hbm, src_hbm, dst_hbm, out_hbm, src_v, dst_v, rows_v,
        zero_v, acc, gsems, ssems):
    cid = lax.axis_index("c")
    sid = lax.axis_index("s")
    wid = sid * NC + cid
    row0 = sid * ROWS_PER_TILE

    pltpu.sync_copy(src_hbm.at[pl.ds(wid * n_chunk, n_chunk)], src_v)
    pltpu.sync_copy(dst_hbm.at[pl.ds(wid * n_chunk, n_chunk)], dst_v)

    _zero_fill(zero_v, F)
    @pl.loop(0, ROWS_PER_TILE, step=16)
    def _(r):
      pltpu.sync_copy(zero_v, acc.at[pl.ds(row0 + r, 16)])
    plsc.subcore_barrier()

    bufs = [rows_v.at[b] for b in range(nbuf)]

    def g_start(i, b):
      pltpu.async_copy(g_hbm.at[src_v.at[i]], bufs[b], gsems[b])

    def g_wait(i, b):
      pltpu.make_async_copy(g_hbm.at[src_v.at[i]], bufs[b], gsems[b]).wait()

    def s_start(i, b):
      pltpu.async_copy(bufs[b], acc.at[dst_v.at[i]], ssems[b], add=True)

    def s_wait(i, b):
      pltpu.make_async_copy(bufs[b], acc.at[dst_v.at[i]], ssems[b]).wait()

    lag = nbuf // 2

    def slot(i, b, static):
      g_wait(i, b)
      s_start(i, b)
      if not static or i - lag >= 0:
        s_wait(i - lag, (b - lag) % nbuf)
      if not static or i + lag <= n_chunk - 1:
        g_start(i + lag, (b + lag) % nbuf)

    loop_end = ((n_chunk - lag) // nbuf) * nbuf
    for i in range(min(lag, n_chunk)):
      g_start(i, i % nbuf)
    for i in range(min(nbuf, loop_end)):
      slot(i, i % nbuf, static=True)

    @pl.loop(nbuf, loop_end, step=nbuf)
    def _(i0):
      for j in range(nbuf):
        slot(i0 + j, j, static=False)

    for i in range(loop_end, n_chunk):
      slot(i, i % nbuf, static=True)
    for i in range(max(0, n_chunk - lag), n_chunk):
      s_wait(i, i % nbuf)

    plsc.subcore_barrier()
    o0 = sid * OUT_PER_TILE
    pltpu.sync_copy(acc.at[pl.ds(o0, OUT_PER_TILE)],
                    out_hbm.at[cid].at[pl.ds(o0, OUT_PER_TILE)])

  return k


C_E = 128
C_HID = 64
_seg_scatter_hid = _make_seg_scatter(HID, C_HID, 4)
_seg_scatter_16 = _make_seg_scatter(16, C_E, 8)


@functools.partial(
    pl.kernel,
    out_type=jax.ShapeDtypeStruct((NC, N, 16), jnp.float32),
    mesh=_mesh,
    compiler_params=pltpu.CompilerParams(use_tc_tiling_on_sc=False),
    scratch_types=[
        pltpu.VMEM((PER_W // C_E, C_E), jnp.int32),
        pltpu.VMEM((C_E, 16), jnp.float32),
        pltpu.VMEM((16, 16), jnp.float32),
        pltpu.VMEM_SHARED((N_PAD, 16), jnp.float32),
        pltpu.SemaphoreType.DMA,
    ],
)
def _degree_kernel(dst_hbm, out_hbm, dst_v, ones_v, zero_v, acc, sem):
  n_chunk = PER_W // C_E
  cid = lax.axis_index("c")
  sid = lax.axis_index("s")
  wid = sid * NC + cid
  row0 = sid * ROWS_PER_TILE

  pltpu.sync_copy(dst_hbm.at[pl.ds(wid * n_chunk, n_chunk)], dst_v)

  @pl.loop(0, C_E)
  def _(r):
    ones_v.at[pl.ds(r, 1), pl.ds(0, 16)][...] = jnp.ones((1, 16), jnp.float32)
  _zero_fill(zero_v, 16)
  @pl.loop(0, ROWS_PER_TILE, step=16)
  def _(r):
    pltpu.sync_copy(zero_v, acc.at[pl.ds(row0 + r, 16)])
  plsc.subcore_barrier()

  @pl.loop(0, 8)
  def _(i):
    pltpu.async_copy(ones_v, acc.at[dst_v.at[i]], sem, add=True)

  @pl.loop(8, n_chunk)
  def _(i):
    pltpu.async_copy(ones_v, acc.at[dst_v.at[i]], sem, add=True)
    pltpu.make_async_copy(ones_v, acc.at[dst_v.at[i - 8]], sem).wait()

  @pl.loop(n_chunk - 8, n_chunk)
  def _(i):
    pltpu.make_async_copy(ones_v, acc.at[dst_v.at[i]], sem).wait()

  plsc.subcore_barrier()
  o0 = sid * OUT_PER_TILE
  pltpu.sync_copy(acc.at[pl.ds(o0, OUT_PER_TILE)],
                  out_hbm.at[cid].at[pl.ds(o0, OUT_PER_TILE)])


ROWS_BLK = 1000
GRID = (N // ROWS_BLK,)

_DOT = dict(dimension_numbers=(((1,), (0,)), ((), ())),
            preferred_element_type=jnp.float32,
            precision=lax.Precision.DEFAULT)


def _selu(x):
  alpha = 1.6732632423543772
  scale = 1.0507009873554805
  return scale * jnp.where(x > 0, x, alpha * (jnp.exp(x) - 1.0))


def _mm_body(x_ref, w_ref, o_ref):
  o_ref[...] = lax.dot_general(x_ref[...], w_ref[...], **_DOT)


def _scale_body(deg_ref, xw_ref, g_ref, dv_ref):
  deg = deg_ref[...]
  dinv = lax.rsqrt(deg[0] + deg[1] + 1.0)
  g_ref[...] = dinv[:, :1] * xw_ref[...]
  dv_ref[...] = dinv


def _layer_body(p_ref, g_ref, dv_ref, b_ref, w_ref, o_ref):
  dinv = dv_ref[...][:, :1]
  a = p_ref[0] + p_ref[1] + g_ref[...]
  h = _selu(dinv * a + b_ref[...])
  o_ref[...] = lax.dot_general(h * dinv, w_ref[...], **_DOT)


def _final_body(p_ref, g_ref, dv_ref, b_ref, o_ref):
  dinv = dv_ref[...][:, :1]
  o = dinv * (p_ref[0] + p_ref[1] + g_ref[...]) + b_ref[...]
  col = lax.broadcasted_iota(jnp.int32, o.shape, 1)
  xm = jnp.where(col < D_OUT, o, -1e30)
  m = jnp.max(xm, axis=1, keepdims=True)
  lse = jnp.log(jnp.sum(jnp.exp(xm - m), axis=1, keepdims=True)) + m
  o_ref[...] = (o - lse)[:, :D_OUT]


def _rows_spec(f):
  return pl.BlockSpec((ROWS_BLK, f), lambda i: (i, 0))


def _pair_spec(f):
  return pl.BlockSpec((2, ROWS_BLK, f), lambda i: (0, i, 0))


def _full_spec(shape):
  return pl.BlockSpec(shape, lambda i: tuple(0 for _ in shape))


def _mm_call(x, w):
  return pl.pallas_call(
      _mm_body, grid=GRID,
      in_specs=[_rows_spec(x.shape[1]), _full_spec(w.shape)],
      out_specs=_rows_spec(w.shape[1]),
      out_shape=jax.ShapeDtypeStruct((N, w.shape[1]), jnp.float32),
  )(x, w)


def _scale_call(degp, xw):
  return pl.pallas_call(
      _scale_body, grid=GRID,
      in_specs=[_pair_spec(16), _rows_spec(HID)],
      out_specs=(_rows_spec(HID), _rows_spec(16)),
      out_shape=(jax.ShapeDtypeStruct((N, HID), jnp.float32),
                 jax.ShapeDtypeStruct((N, 16), jnp.float32)),
  )(degp, xw)


def _layer_call(p, g, dv, b, w):
  f_in, f_out = w.shape
  return pl.pallas_call(
      _layer_body, grid=GRID,
      in_specs=[_pair_spec(f_in), _rows_spec(f_in), _rows_spec(16),
                _full_spec((1, f_in)), _full_spec(w.shape)],
      out_specs=_rows_spec(f_out),
      out_shape=jax.ShapeDtypeStruct((N, f_out), jnp.float32),
  )(p, g, dv, b, w)


def _final_call(p, g, dv, b):
  return pl.pallas_call(
      _final_body, grid=GRID,
      in_specs=[_pair_spec(16), _rows_spec(16), _rows_spec(16),
                _full_spec((1, 16))],
      out_specs=_rows_spec(D_OUT),
      out_shape=jax.ShapeDtypeStruct((N, D_OUT), jnp.float32),
  )(p, g, dv, b)


def kernel(x, edge_index, W1, b1, W2, b2, W3, b3):
  src = edge_index[0]
  dst = edge_index[1]
  pad_i = jnp.arange(E_PAD - E, dtype=jnp.int32) % PAD_SPREAD
  src_p = jnp.concatenate([src, pad_i])
  dst_p = jnp.concatenate([dst, N + pad_i])
  src2 = src_p.reshape(E_PAD // C_E, C_E)
  dst2 = dst_p.reshape(E_PAD // C_E, C_E)
  src2h = src_p.reshape(E_PAD // C_HID, C_HID)
  dst2h = dst_p.reshape(E_PAD // C_HID, C_HID)
  w3p = jnp.pad(W3, ((0, 0), (0, 16 - D_OUT)))
  b1r = b1.reshape(1, HID)
  b2r = b2.reshape(1, HID)
  b3r = jnp.pad(b3, (0, 16 - D_OUT)).reshape(1, 16)

  degp = _degree_kernel(dst2)
  xw1 = _mm_call(x, W1)
  g1, dv = _scale_call(degp, xw1)
  p1 = _seg_scatter_hid(g1, src2h, dst2h)
  g2 = _layer_call(p1, g1, dv, b1r, W2)
  p2 = _seg_scatter_hid(g2, src2h, dst2h)
  g3 = _layer_call(p2, g2, dv, b2r, w3p)
  p3 = _seg_scatter_16(g3, src2, dst2)
  return _final_call(p3, g3, dv, b3r)


# speedup vs baseline: 22.9000x; 1.0296x over previous
import functools

import jax
import jax.numpy as jnp
from jax import lax
from jax.experimental import pallas as pl
from jax.experimental.pallas import tpu as pltpu
from jax.experimental.pallas import tpu_sc as plsc

N = 10000
E = 160000
D_IN = 256
HID = 128
D_OUT = 12

NC, NS = 2, 16
NW = NC * NS
N_PAD = 10240
E_PAD = 163840
PER_W = E_PAD // NW
ROWS_PER_TILE = N_PAD // NS
OUT_PER_TILE = N // NS
PAD_SPREAD = 240

_mesh = plsc.VectorSubcoreMesh(core_axis_name="c", subcore_axis_name="s",
                               num_cores=NC, num_subcores=NS)


def _zero_fill(zero_v, F):
  @pl.loop(0, 16)
  def _(r):
    @pl.loop(0, F, step=16)
    def _(j):
      zero_v.at[pl.ds(r, 1), pl.ds(j, 16)][...] = jnp.zeros((1, 16), jnp.float32)


def _make_seg_scatter(F, C, nbuf):
  n_chunk = PER_W // C
  cparams = pltpu.CompilerParams(use_tc_tiling_on_sc=False)

  @functools.partial(
      pl.kernel,
      compiler_params=cparams,
      out_type=jax.ShapeDtypeStruct((NC, N, F), jnp.float32),
      mesh=_mesh,
      scratch_types=[
          pltpu.VMEM((n_chunk, C), jnp.int32),
          pltpu.VMEM((n_chunk, C), jnp.int32),
          pltpu.VMEM((nbuf, C, F), jnp.float32),
          pltpu.VMEM((16, F), jnp.float32),
          pltpu.VMEM_SHARED((N_PAD, F), jnp.float32),
          [pltpu.SemaphoreType.DMA] * nbuf,
          [pltpu.SemaphoreType.DMA] * nbuf,
      ],
  )
  def k(g_hbm, src_hbm, dst_hbm, out_hbm, src_v, dst_v, rows_v,
        zero_v, acc, gsems, ssems):
    cid = lax.axis_index("c")
    sid = lax.axis_index("s")
    wid = sid * NC + cid
    row0 = sid * ROWS_PER_TILE

    pltpu.sync_copy(src_hbm.at[pl.ds(wid * n_chunk, n_chunk)], src_v)
    pltpu.sync_copy(dst_hbm.at[pl.ds(wid * n_chunk, n_chunk)], dst_v)

    _zero_fill(zero_v, F)
    @pl.loop(0, ROWS_PER_TILE, step=16)
    def _(r):
      pltpu.sync_copy(zero_v, acc.at[pl.ds(row0 + r, 16)])
    plsc.subcore_barrier()

    bufs = [rows_v.at[b] for b in range(nbuf)]

    def g_start(i, b):
      pltpu.async_copy(g_hbm.at[src_v.at[i]], bufs[b], gsems[b])

    def g_wait(i, b):
      pltpu.make_async_copy(g_hbm.at[src_v.at[i]], bufs[b], gsems[b]).wait()

    def s_start(i, b):
      pltpu.async_copy(bufs[b], acc.at[dst_v.at[i]], ssems[b], add=True)

    def s_wait(i, b):
      pltpu.make_async_copy(bufs[b], acc.at[dst_v.at[i]], ssems[b]).wait()

    lag = nbuf // 2

    def slot(i, b, static):
      g_wait(i, b)
      s_start(i, b)
      if not static or i - lag >= 0:
        s_wait(i - lag, (b - lag) % nbuf)
      if not static or i + lag <= n_chunk - 1:
        g_start(i + lag, (b + lag) % nbuf)

    loop_end = ((n_chunk - lag) // nbuf) * nbuf
    for i in range(min(lag, n_chunk)):
      g_start(i, i % nbuf)
    for i in range(min(nbuf, loop_end)):
      slot(i, i % nbuf, static=True)

    @pl.loop(nbuf, loop_end, step=nbuf)
    def _(i0):
      for j in range(nbuf):
        slot(i0 + j, j, static=False)

    for i in range(loop_end, n_chunk):
      slot(i, i % nbuf, static=True)
    for i in range(max(0, n_chunk - lag), n_chunk):
      s_wait(i, i % nbuf)

    plsc.subcore_barrier()
    o0 = sid * OUT_PER_TILE
    pltpu.sync_copy(acc.at[pl.ds(o0, OUT_PER_TILE)],
                    out_hbm.at[cid].at[pl.ds(o0, OUT_PER_TILE)])

  return k


C_E = 128
C_HID = 64
_seg_scatter_hid = _make_seg_scatter(HID, C_HID, 4)
_seg_scatter_16 = _make_seg_scatter(16, C_E, 8)


@functools.partial(
    pl.kernel,
    out_type=jax.ShapeDtypeStruct((NC, N, 16), jnp.float32),
    mesh=_mesh,
    compiler_params=pltpu.CompilerParams(use_tc_tiling_on_sc=False),
    scratch_types=[
        pltpu.VMEM((PER_W // C_E, C_E), jnp.int32),
        pltpu.VMEM((C_E, 16), jnp.float32),
        pltpu.VMEM((16, 16), jnp.float32),
        pltpu.VMEM_SHARED((N_PAD, 16), jnp.float32),
        pltpu.SemaphoreType.DMA,
    ],
)
def _degree_kernel(dst_hbm, out_hbm, dst_v, ones_v, zero_v, acc, sem):
  n_chunk = PER_W // C_E
  cid = lax.axis_index("c")
  sid = lax.axis_index("s")
  wid = sid * NC + cid
  row0 = sid * ROWS_PER_TILE

  pltpu.sync_copy(dst_hbm.at[pl.ds(wid * n_chunk, n_chunk)], dst_v)

  @pl.loop(0, C_E)
  def _(r):
    ones_v.at[pl.ds(r, 1), pl.ds(0, 16)][...] = jnp.ones((1, 16), jnp.float32)
  _zero_fill(zero_v, 16)
  @pl.loop(0, ROWS_PER_TILE, step=16)
  def _(r):
    pltpu.sync_copy(zero_v, acc.at[pl.ds(row0 + r, 16)])
  plsc.subcore_barrier()

  @pl.loop(0, 8)
  def _(i):
    pltpu.async_copy(ones_v, acc.at[dst_v.at[i]], sem, add=True)

  @pl.loop(8, n_chunk)
  def _(i):
    pltpu.async_copy(ones_v, acc.at[dst_v.at[i]], sem, add=True)
    pltpu.make_async_copy(ones_v, acc.at[dst_v.at[i - 8]], sem).wait()

  @pl.loop(n_chunk - 8, n_chunk)
  def _(i):
    pltpu.make_async_copy(ones_v, acc.at[dst_v.at[i]], sem).wait()

  plsc.subcore_barrier()
  o0 = sid * OUT_PER_TILE
  pltpu.sync_copy(acc.at[pl.ds(o0, OUT_PER_TILE)],
                  out_hbm.at[cid].at[pl.ds(o0, OUT_PER_TILE)])


ROWS_BLK = 2000
GRID = (N // ROWS_BLK,)

_DOT = dict(dimension_numbers=(((1,), (0,)), ((), ())),
            preferred_element_type=jnp.float32,
            precision=lax.Precision.DEFAULT)


def _selu(x):
  alpha = 1.6732632423543772
  scale = 1.0507009873554805
  return scale * jnp.where(x > 0, x, alpha * (jnp.exp(x) - 1.0))


def _mm_body(x_ref, w_ref, o_ref):
  o_ref[...] = lax.dot_general(x_ref[...], w_ref[...], **_DOT)


def _scale_body(deg_ref, xw_ref, g_ref, dv_ref):
  deg = deg_ref[...]
  dinv = lax.rsqrt(deg[0] + deg[1] + 1.0)
  g_ref[...] = dinv[:, :1] * xw_ref[...]
  dv_ref[...] = dinv


def _layer_body(p_ref, g_ref, dv_ref, b_ref, w_ref, o_ref):
  dinv = dv_ref[...][:, :1]
  a = p_ref[0] + p_ref[1] + g_ref[...]
  h = _selu(dinv * a + b_ref[...])
  o_ref[...] = lax.dot_general(h * dinv, w_ref[...], **_DOT)


def _final_body(p_ref, g_ref, dv_ref, b_ref, o_ref):
  dinv = dv_ref[...][:, :1]
  o = dinv * (p_ref[0] + p_ref[1] + g_ref[...]) + b_ref[...]
  col = lax.broadcasted_iota(jnp.int32, o.shape, 1)
  xm = jnp.where(col < D_OUT, o, -1e30)
  m = jnp.max(xm, axis=1, keepdims=True)
  lse = jnp.log(jnp.sum(jnp.exp(xm - m), axis=1, keepdims=True)) + m
  o_ref[...] = (o - lse)[:, :D_OUT]


def _rows_spec(f):
  return pl.BlockSpec((ROWS_BLK, f), lambda i: (i, 0))


def _pair_spec(f):
  return pl.BlockSpec((2, ROWS_BLK, f), lambda i: (0, i, 0))


def _full_spec(shape):
  return pl.BlockSpec(shape, lambda i: tuple(0 for _ in shape))


def _mm_call(x, w):
  return pl.pallas_call(
      _mm_body, grid=GRID,
      in_specs=[_rows_spec(x.shape[1]), _full_spec(w.shape)],
      out_specs=_rows_spec(w.shape[1]),
      out_shape=jax.ShapeDtypeStruct((N, w.shape[1]), jnp.float32),
  )(x, w)


def _scale_call(degp, xw):
  return pl.pallas_call(
      _scale_body, grid=GRID,
      in_specs=[_pair_spec(16), _rows_spec(HID)],
      out_specs=(_rows_spec(HID), _rows_spec(16)),
      out_shape=(jax.ShapeDtypeStruct((N, HID), jnp.float32),
                 jax.ShapeDtypeStruct((N, 16), jnp.float32)),
  )(degp, xw)


def _layer_call(p, g, dv, b, w):
  f_in, f_out = w.shape
  return pl.pallas_call(
      _layer_body, grid=GRID,
      in_specs=[_pair_spec(f_in), _rows_spec(f_in), _rows_spec(16),
                _full_spec((1, f_in)), _full_spec(w.shape)],
      out_specs=_rows_spec(f_out),
      out_shape=jax.ShapeDtypeStruct((N, f_out), jnp.float32),
  )(p, g, dv, b, w)


def _final_call(p, g, dv, b):
  return pl.pallas_call(
      _final_body, grid=GRID,
      in_specs=[_pair_spec(16), _rows_spec(16), _rows_spec(16),
                _full_spec((1, 16))],
      out_specs=_rows_spec(D_OUT),
      out_shape=jax.ShapeDtypeStruct((N, D_OUT), jnp.float32),
  )(p, g, dv, b)


def kernel(x, edge_index, W1, b1, W2, b2, W3, b3):
  src = edge_index[0]
  dst = edge_index[1]
  pad_i = jnp.arange(E_PAD - E, dtype=jnp.int32) % PAD_SPREAD
  src_p = jnp.concatenate([src, pad_i])
  dst_p = jnp.concatenate([dst, N + pad_i])
  src2 = src_p.reshape(E_PAD // C_E, C_E)
  dst2 = dst_p.reshape(E_PAD // C_E, C_E)
  src2h = src_p.reshape(E_PAD // C_HID, C_HID)
  dst2h = dst_p.reshape(E_PAD // C_HID, C_HID)
  w3p = jnp.pad(W3, ((0, 0), (0, 16 - D_OUT)))
  b1r = b1.reshape(1, HID)
  b2r = b2.reshape(1, HID)
  b3r = jnp.pad(b3, (0, 16 - D_OUT)).reshape(1, 16)

  degp = _degree_kernel(dst2)
  xw1 = _mm_call(x, W1)
  g1, dv = _scale_call(degp, xw1)
  p1 = _seg_scatter_hid(g1, src2h, dst2h)
  g2 = _layer_call(p1, g1, dv, b1r, W2)
  p2 = _seg_scatter_hid(g2, src2h, dst2h)
  g3 = _layer_call(p2, g2, dv, b2r, w3p)
  p3 = _seg_scatter_16(g3, src2, dst2)
  return _final_call(p3, g3, dv, b3r)
